# Initial kernel scaffold; baseline (speedup 1.0000x reference)
import jax
import jax.numpy as jnp
from jax.experimental import pallas as pl


def kernel(x, edge_index, eps0, W1_0, b1_0, W2_0, b2_0, eps1, W1_1, b1_1, W2_1, b2_1, Wf1, bf1, Wf2, bf2):
    raise NotImplementedError("write your pallas kernel here")


# speedup vs baseline: 5.0006x; 5.0006x over previous
import functools

import jax
import jax.numpy as jnp
from jax import lax
from jax.experimental import pallas as pl
from jax.experimental.pallas import tpu as pltpu
from jax.experimental.pallas import tpu_sc as plsc

N = 10000
E = 320000
D = 128

NC = 2
NS = 16
NW = NC * NS
N_PAD = 10240
ROWS_PER_TILE = N_PAD // NS
E_PER_W = E // NW
CHUNK = 80
N_CHUNKS = E_PER_W // CHUNK


def _spmm_partials(x_pad, src, dst, zeros_block):
    mesh = plsc.VectorSubcoreMesh(core_axis_name="c", subcore_axis_name="s")

    @functools.partial(
        pl.kernel,
        mesh=mesh,
        out_type=jax.ShapeDtypeStruct((NC * N_PAD, D), jnp.float32),
        scratch_types=[
            pltpu.VMEM((CHUNK,), jnp.int32),
            pltpu.VMEM((CHUNK,), jnp.int32),
            pltpu.VMEM((CHUNK, D), jnp.float32),
            pltpu.VMEM_SHARED((N_PAD, D), jnp.float32),
            pltpu.SemaphoreType.DMA,
        ],
    )
    def spmm(x_hbm, src_hbm, dst_hbm, zb_hbm, out_hbm, sidx, didx, rows, acc, sem):
        c = lax.axis_index("c")
        s = lax.axis_index("s")
        wid = s * NC + c

        row0 = pl.multiple_of(s * ROWS_PER_TILE, ROWS_PER_TILE)
        pltpu.sync_copy(zb_hbm, acc.at[pl.ds(row0, ROWS_PER_TILE)])
        plsc.subcore_barrier()

        base = wid * E_PER_W

        def body(j, carry):
            off = pl.multiple_of(base + j * CHUNK, CHUNK)
            pltpu.sync_copy(src_hbm.at[pl.ds(off, CHUNK)], sidx)
            pltpu.sync_copy(dst_hbm.at[pl.ds(off, CHUNK)], didx)
            pltpu.async_copy(x_hbm.at[sidx], rows, sem).wait()
            pltpu.sync_copy(rows, acc.at[didx], add=True)
            return carry

        lax.fori_loop(0, N_CHUNKS, body, 0)
        plsc.subcore_barrier()

        out0 = pl.multiple_of(c * N_PAD + s * ROWS_PER_TILE, ROWS_PER_TILE)
        pltpu.sync_copy(acc.at[pl.ds(row0, ROWS_PER_TILE)],
                        out_hbm.at[pl.ds(out0, ROWS_PER_TILE)])

    return spmm(x_pad, src, dst, zeros_block)


_BM = 640


def _mlp0_body(p_ref, x_ref, eps_ref, w1_ref, b1_ref, w2_ref, b2_ref, o_ref):
    h = p_ref[0] + p_ref[1] + (1.0 + eps_ref[0, 0]) * x_ref[...]
    h = jnp.dot(h, w1_ref[...], preferred_element_type=jnp.float32) + b1_ref[...]
    h = jnp.maximum(h, 0.0)
    o_ref[...] = jnp.dot(h, w2_ref[...], preferred_element_type=jnp.float32) + b2_ref[...]


def _mlp_layer(p, x_pad, eps, w1, b1, w2, b2):
    grid = (N_PAD // _BM,)
    wspec = pl.BlockSpec((D, D), lambda i: (0, 0))
    bspec = pl.BlockSpec((1, D), lambda i: (0, 0))
    return pl.pallas_call(
        _mlp0_body,
        grid=grid,
        in_specs=[
            pl.BlockSpec((2, _BM, D), lambda i: (0, i, 0)),
            pl.BlockSpec((_BM, D), lambda i: (i, 0)),
            pl.BlockSpec((1, 1), lambda i: (0, 0)),
            wspec, bspec, wspec, bspec,
        ],
        out_specs=pl.BlockSpec((_BM, D), lambda i: (i, 0)),
        out_shape=jax.ShapeDtypeStruct((N_PAD, D), jnp.float32),
    )(p, x_pad, eps, w1, b1, w2, b2)


def _mlp1_final_body(p_ref, x_ref, eps_ref, w1_ref, b1_ref, w2_ref, b2_ref,
                     wf1_ref, bf1_ref, wf2_ref, bf2_ref, o_ref):
    h = p_ref[0] + p_ref[1] + (1.0 + eps_ref[0, 0]) * x_ref[...]
    h = jnp.dot(h, w1_ref[...], preferred_element_type=jnp.float32) + b1_ref[...]
    h = jnp.maximum(h, 0.0)
    h = jnp.dot(h, w2_ref[...], preferred_element_type=jnp.float32) + b2_ref[...]
    h = jnp.dot(h, wf1_ref[...], preferred_element_type=jnp.float32) + bf1_ref[...]
    h = jnp.maximum(h, 0.0)
    o_ref[...] = jnp.dot(h, wf2_ref[...], preferred_element_type=jnp.float32) + bf2_ref[...]


def _mlp1_final(p, h_pad, eps, w1, b1, w2, b2, wf1, bf1, wf2, bf2):
    grid = (N_PAD // _BM,)
    wspec = pl.BlockSpec((D, D), lambda i: (0, 0))
    bspec = pl.BlockSpec((1, D), lambda i: (0, 0))
    return pl.pallas_call(
        _mlp1_final_body,
        grid=grid,
        in_specs=[
            pl.BlockSpec((2, _BM, D), lambda i: (0, i, 0)),
            pl.BlockSpec((_BM, D), lambda i: (i, 0)),
            pl.BlockSpec((1, 1), lambda i: (0, 0)),
            wspec, bspec, wspec, bspec,
            wspec, bspec, wspec, bspec,
        ],
        out_specs=pl.BlockSpec((_BM, D), lambda i: (i, 0)),
        out_shape=jax.ShapeDtypeStruct((N_PAD, D), jnp.float32),
    )(p, h_pad, eps, w1, b1, w2, b2, wf1, bf1, wf2, bf2)


def kernel(x, edge_index, eps0, W1_0, b1_0, W2_0, b2_0, eps1, W1_1, b1_1,
           W2_1, b2_1, Wf1, bf1, Wf2, bf2):
    src = edge_index[0]
    dst = edge_index[1]
    x_pad = jnp.pad(x, ((0, N_PAD - N), (0, 0)))
    zb = jnp.zeros((ROWS_PER_TILE, D), jnp.float32)
    eps0_2d = jnp.reshape(eps0, (1, 1))
    eps1_2d = jnp.reshape(eps1, (1, 1))

    p0 = _spmm_partials(x_pad, src, dst, zb).reshape(NC, N_PAD, D)
    h1 = _mlp_layer(p0, x_pad, eps0_2d, W1_0, jnp.reshape(b1_0, (1, D)),
                    W2_0, jnp.reshape(b2_0, (1, D)))
    p1 = _spmm_partials(h1, src, dst, zb).reshape(NC, N_PAD, D)
    out = _mlp1_final(p1, h1, eps1_2d, W1_1, jnp.reshape(b1_1, (1, D)),
                      W2_1, jnp.reshape(b2_1, (1, D)),
                      Wf1, jnp.reshape(bf1, (1, D)),
                      Wf2, jnp.reshape(bf2, (1, D)))
    return out[:N]
